# baseline (speedup 1.0000x reference)
import functools

import jax
import jax.numpy as jnp
from jax import lax
from jax.experimental import pallas as pl
from jax.experimental.pallas import tpu as pltpu
from jax.experimental.pallas import tpu_sc as plsc

NUM_EXPERTS = 64
TOPK = 8
TOKEN_BLOCK = 512
NUM_TOKENS = 8192
NUM_CHUNKS = 4
CHUNK = NUM_TOKENS // NUM_CHUNKS
NC = 2
NS = 16
L = 16
TOK_PER_W = CHUNK // (NC * NS)


def _softmax_kernel(x_ref, w_ref, p_ref):
    logits = jax.lax.dot_general(
        w_ref[...], x_ref[...],
        dimension_numbers=(((1,), (1,)), ((), ())),
        preferred_element_type=jnp.float32,
    )
    col_max = jnp.max(logits, axis=0, keepdims=True)
    e = jnp.exp(logits - col_max)
    p = e / jnp.sum(e, axis=0, keepdims=True)
    p_ref[...] = p.T


def _tc_softmax(x, weight):
    num_tokens, hidden = x.shape
    grid = (num_tokens // TOKEN_BLOCK,)
    return pl.pallas_call(
        _softmax_kernel,
        grid=grid,
        in_specs=[
            pl.BlockSpec((TOKEN_BLOCK, hidden), lambda i: (i, 0)),
            pl.BlockSpec((NUM_EXPERTS, hidden), lambda i: (0, 0)),
        ],
        out_specs=pl.BlockSpec((TOKEN_BLOCK, NUM_EXPERTS), lambda i: (i, 0)),
        out_shape=jax.ShapeDtypeStruct((num_tokens, NUM_EXPERTS), jnp.float32),
    )(x, weight)


def _sc_topk_body(p_hbm, probs_hbm, map_hbm, chunk, oprobs, omap):
    wid = lax.axis_index("s") * NC + lax.axis_index("c")
    base = wid * TOK_PER_W
    pltpu.sync_copy(p_hbm.at[pl.ds(base, TOK_PER_W)], chunk)

    lanes = lax.broadcasted_iota(jnp.int32, (L,), 0)

    def body(t, _):
        s = [chunk[t, pl.ds(k * L, L)] for k in range(4)]
        ss = [plsc.sort_key_val(sk, sk)[0] for sk in s]
        u = jnp.maximum(ss[0], lax.rev(ss[1], (0,)))
        v = jnp.maximum(ss[2], lax.rev(ss[3], (0,)))
        us = plsc.sort_key_val(u, u)[0]
        vs = plsc.sort_key_val(v, v)[0]
        w = jnp.maximum(us, lax.rev(vs, (0,)))
        ws = plsc.sort_key_val(w, w)[0]
        thr = jnp.min(jnp.where(lanes >= TOPK, ws, jnp.inf), axis=0)
        thr = jnp.broadcast_to(thr, (L,))
        gt = [sk > thr for sk in s]
        cnt_gt = plsc.all_reduce_population_count(gt[0])
        for k in range(1, 4):
            cnt_gt = cnt_gt + plsc.all_reduce_population_count(gt[k])
        need_ties = TOPK - cnt_gt
        prior = jnp.zeros((L,), jnp.int32)
        for k in range(4):
            tie = s[k] == thr
            cum = jnp.cumsum(tie.astype(jnp.int32))
            sel_tie = jnp.logical_and(tie, (prior + cum) <= need_ties)
            mask = jnp.logical_or(gt[k], sel_tie)
            oprobs[t, pl.ds(k * L, L)] = jnp.where(mask, s[k], 0.0)
            omap[t, pl.ds(k * L, L)] = mask.astype(jnp.float32)
            prior = prior + plsc.all_reduce_population_count(tie)
        return _

    lax.fori_loop(0, TOK_PER_W, body, None)
    pltpu.sync_copy(oprobs, probs_hbm.at[pl.ds(base, TOK_PER_W)])
    pltpu.sync_copy(omap, map_hbm.at[pl.ds(base, TOK_PER_W)])


_sc_topk = functools.partial(
    pl.kernel,
    out_type=[
        jax.ShapeDtypeStruct((CHUNK, NUM_EXPERTS), jnp.float32),
        jax.ShapeDtypeStruct((CHUNK, NUM_EXPERTS), jnp.float32),
    ],
    scratch_types=[
        pltpu.VMEM((TOK_PER_W, NUM_EXPERTS), jnp.float32),
        pltpu.VMEM((TOK_PER_W, NUM_EXPERTS), jnp.float32),
        pltpu.VMEM((TOK_PER_W, NUM_EXPERTS), jnp.float32),
    ],
    mesh=plsc.VectorSubcoreMesh(core_axis_name="c", subcore_axis_name="s"),
    compiler_params=pltpu.CompilerParams(needs_layout_passes=False),
)(_sc_topk_body)


@jax.jit
def kernel(x, weight):
    outs = []
    for i in range(NUM_CHUNKS):
        p = _tc_softmax(x[i * CHUNK:(i + 1) * CHUNK], weight)
        outs.append(_sc_topk(p))
    probs = jnp.concatenate([o[0] for o in outs], axis=0)
    rmap = jnp.concatenate([o[1] for o in outs], axis=0)
    return probs, rmap.astype(jnp.bool_)


# speedup vs baseline: 3.1469x; 3.1469x over previous
import jax
import jax.numpy as jnp
from jax.experimental import pallas as pl

NUM_EXPERTS = 64
TOPK = 8
TOKEN_BLOCK = 1024


def _router_kernel(x_ref, w_ref, probs_ref, map_ref):
    logits = jax.lax.dot_general(
        w_ref[...], x_ref[...],
        dimension_numbers=(((1,), (1,)), ((), ())),
        preferred_element_type=jnp.float32,
    )
    col_max = jnp.max(logits, axis=0, keepdims=True)
    e = jnp.exp(logits - col_max)
    p = e / jnp.sum(e, axis=0, keepdims=True)

    neg_inf = jnp.float32(-jnp.inf)

    v = p
    for _ in range(TOPK):
        mx = jnp.max(v, axis=0, keepdims=True)
        v = jnp.where(v == mx, neg_inf, v)
    mask = v != p
    cnt = jnp.sum(mask.astype(jnp.float32))
    exact = cnt == jnp.float32(TOPK * TOKEN_BLOCK)

    @pl.when(exact)
    def _():
        probs_ref[...] = jnp.where(mask, p, 0.0).T
        map_ref[...] = mask.astype(jnp.float32).T

    @pl.when(jnp.logical_not(exact))
    def _():
        rows = jax.lax.broadcasted_iota(jnp.int32, logits.shape, 0)
        v2 = p
        m2 = jnp.zeros(logits.shape, dtype=jnp.bool_)
        for _ in range(TOPK):
            mx = jnp.max(v2, axis=0, keepdims=True)
            at_max = v2 == mx
            first = jnp.min(
                jnp.where(at_max, rows, NUM_EXPERTS), axis=0, keepdims=True
            )
            sel = rows == first
            m2 = jnp.logical_or(m2, sel)
            v2 = jnp.where(sel, neg_inf, v2)
        probs_ref[...] = jnp.where(m2, p, 0.0).T
        map_ref[...] = m2.astype(jnp.float32).T


@jax.jit
def kernel(x, weight):
    num_tokens, hidden = x.shape
    grid = (num_tokens // TOKEN_BLOCK,)
    probs, rmap = pl.pallas_call(
        _router_kernel,
        grid=grid,
        in_specs=[
            pl.BlockSpec((TOKEN_BLOCK, hidden), lambda i: (i, 0)),
            pl.BlockSpec((NUM_EXPERTS, hidden), lambda i: (0, 0)),
        ],
        out_specs=[
            pl.BlockSpec((TOKEN_BLOCK, NUM_EXPERTS), lambda i: (i, 0)),
            pl.BlockSpec((TOKEN_BLOCK, NUM_EXPERTS), lambda i: (i, 0)),
        ],
        out_shape=[
            jax.ShapeDtypeStruct((num_tokens, NUM_EXPERTS), jnp.float32),
            jax.ShapeDtypeStruct((num_tokens, NUM_EXPERTS), jnp.float32),
        ],
    )(x, weight)
    return probs, rmap.astype(jnp.bool_)


# speedup vs baseline: 3.1689x; 1.0070x over previous
import jax
import jax.numpy as jnp
from jax.experimental import pallas as pl

NUM_EXPERTS = 64
TOPK = 8
TOKEN_BLOCK = 1024
SUB = 256
NSUB = TOKEN_BLOCK // SUB


def _epilogue(logits, probs_ref, map_ref, s):
    col_max = jnp.max(logits, axis=0, keepdims=True)
    e = jnp.exp(logits - col_max)
    p = e / jnp.sum(e, axis=0, keepdims=True)

    neg_inf = jnp.float32(-jnp.inf)
    rows = jax.lax.broadcasted_iota(jnp.int32, logits.shape, 0)
    v = p
    mask = jnp.zeros(logits.shape, dtype=jnp.bool_)
    for _ in range(TOPK):
        mx = jnp.max(v, axis=0, keepdims=True)
        at_max = v == mx
        first = jnp.min(jnp.where(at_max, rows, NUM_EXPERTS), axis=0,
                        keepdims=True)
        sel = rows == first
        mask = jnp.logical_or(mask, sel)
        v = jnp.where(sel, neg_inf, v)

    probs_ref[pl.ds(s * SUB, SUB), :] = jnp.where(mask, p, 0.0).T
    map_ref[pl.ds(s * SUB, SUB), :] = mask.astype(jnp.float32).T


def _router_kernel(x_ref, w_ref, probs_ref, map_ref):
    w = w_ref[...]

    def mm(s):
        return jax.lax.dot_general(
            w, x_ref[pl.ds(s * SUB, SUB), :],
            dimension_numbers=(((1,), (1,)), ((), ())),
            preferred_element_type=jnp.float32,
        )

    prev = mm(0)
    for s in range(1, NSUB):
        cur = mm(s)
        _epilogue(prev, probs_ref, map_ref, s - 1)
        prev = cur
    _epilogue(prev, probs_ref, map_ref, NSUB - 1)


@jax.jit
def kernel(x, weight):
    num_tokens, hidden = x.shape
    grid = (num_tokens // TOKEN_BLOCK,)
    probs, rmap = pl.pallas_call(
        _router_kernel,
        grid=grid,
        in_specs=[
            pl.BlockSpec((TOKEN_BLOCK, hidden), lambda i: (i, 0)),
            pl.BlockSpec((NUM_EXPERTS, hidden), lambda i: (0, 0)),
        ],
        out_specs=[
            pl.BlockSpec((TOKEN_BLOCK, NUM_EXPERTS), lambda i: (i, 0)),
            pl.BlockSpec((TOKEN_BLOCK, NUM_EXPERTS), lambda i: (i, 0)),
        ],
        out_shape=[
            jax.ShapeDtypeStruct((num_tokens, NUM_EXPERTS), jnp.float32),
            jax.ShapeDtypeStruct((num_tokens, NUM_EXPERTS), jnp.float32),
        ],
    )(x, weight)
    return probs, rmap.astype(jnp.bool_)


# speedup vs baseline: 3.1949x; 1.0082x over previous
import jax
import jax.numpy as jnp
from jax.experimental import pallas as pl

NUM_EXPERTS = 64
TOPK = 8
TOKEN_BLOCK = 1024
SUB = 256
NSUB = TOKEN_BLOCK // SUB


def _epilogue(logits, probs_ref, map_ref, s):
    col_max = jnp.max(logits, axis=0, keepdims=True)
    e = jnp.exp(logits - col_max)
    p = e / jnp.sum(e, axis=0, keepdims=True)

    neg_inf = jnp.float32(-jnp.inf)
    rows = jax.lax.broadcasted_iota(jnp.int32, logits.shape, 0)
    v = p
    mask = jnp.zeros(logits.shape, dtype=jnp.bool_)
    for _ in range(TOPK):
        mx = jnp.max(v, axis=0, keepdims=True)
        at_max = v == mx
        first = jnp.min(jnp.where(at_max, rows, NUM_EXPERTS), axis=0,
                        keepdims=True)
        sel = rows == first
        mask = jnp.logical_or(mask, sel)
        v = jnp.where(sel, neg_inf, v)

    probs_ref[pl.ds(s * SUB, SUB), :] = jnp.where(mask, p, 0.0).T
    map_ref[pl.ds(s * SUB, SUB), :] = mask.T.astype(jnp.int8)


def _router_kernel(x_ref, w_ref, probs_ref, map_ref):
    w = w_ref[...]

    def mm(s):
        return jax.lax.dot_general(
            w, x_ref[pl.ds(s * SUB, SUB), :],
            dimension_numbers=(((1,), (1,)), ((), ())),
            preferred_element_type=jnp.float32,
        )

    prev = mm(0)
    for s in range(1, NSUB):
        cur = mm(s)
        _epilogue(prev, probs_ref, map_ref, s - 1)
        prev = cur
    _epilogue(prev, probs_ref, map_ref, NSUB - 1)


@jax.jit
def kernel(x, weight):
    num_tokens, hidden = x.shape
    grid = (num_tokens // TOKEN_BLOCK,)
    probs, rmap = pl.pallas_call(
        _router_kernel,
        grid=grid,
        in_specs=[
            pl.BlockSpec((TOKEN_BLOCK, hidden), lambda i: (i, 0)),
            pl.BlockSpec((NUM_EXPERTS, hidden), lambda i: (0, 0)),
        ],
        out_specs=[
            pl.BlockSpec((TOKEN_BLOCK, NUM_EXPERTS), lambda i: (i, 0)),
            pl.BlockSpec((TOKEN_BLOCK, NUM_EXPERTS), lambda i: (i, 0)),
        ],
        out_shape=[
            jax.ShapeDtypeStruct((num_tokens, NUM_EXPERTS), jnp.float32),
            jax.ShapeDtypeStruct((num_tokens, NUM_EXPERTS), jnp.int8),
        ],
    )(x, weight)
    return probs, rmap.astype(jnp.bool_)
